# baseline (speedup 1.0000x reference)
import functools

import jax
import jax.numpy as jnp
from jax import lax
from jax.experimental import pallas as pl
from jax.experimental.pallas import tpu as pltpu
from jax.experimental.pallas import tpu_sc as plsc


_CHUNK = 64


def _sc_gather_pair(table_s, table_e, idx_s, idx_e):
    n = idx_s.shape[0]
    h = table_s.shape[1]
    info = plsc.get_sparse_core_info()
    nc, ns = info.num_cores, info.num_subcores
    nw = nc * ns
    per_w = n // nw
    nchunks = per_w // _CHUNK
    mesh = plsc.VectorSubcoreMesh(core_axis_name="core", subcore_axis_name="subcore")

    @functools.partial(
        pl.kernel,
        out_type=(
            jax.ShapeDtypeStruct((n, h), jnp.float32),
            jax.ShapeDtypeStruct((n, h), jnp.float32),
        ),
        mesh=mesh,
        scratch_types=[
            pltpu.VMEM((per_w,), jnp.int32),
            pltpu.VMEM((_CHUNK, h), jnp.float32),
            pltpu.SemaphoreType.DMA,
        ],
    )
    def gather_kernel(ts_hbm, te_hbm, is_hbm, ie_hbm, os_hbm, oe_hbm,
                      idx_v, rows_v, sem):
        wid = lax.axis_index("subcore") * nc + lax.axis_index("core")
        base = wid * per_w

        def one_table(t_hbm, i_hbm, o_hbm):
            pltpu.sync_copy(i_hbm.at[pl.ds(base, per_w)], idx_v)

            @pl.loop(0, nchunks)
            def _(c):
                off = c * _CHUNK
                pltpu.async_copy(
                    t_hbm.at[idx_v.at[pl.ds(off, _CHUNK)]], rows_v, sem
                ).wait()
                pltpu.sync_copy(rows_v, o_hbm.at[pl.ds(base + off, _CHUNK)])

        one_table(ts_hbm, is_hbm, os_hbm)
        one_table(te_hbm, ie_hbm, oe_hbm)

    return gather_kernel(table_s, table_e, idx_s, idx_e)


def _fold_body(w2_ref, wpb_ref, b2_ref, bp_ref, w2p_ref, bpr_ref):
    w2p_ref[...] = jnp.dot(
        w2_ref[...], wpb_ref[...], preferred_element_type=jnp.float32
    )
    bpr_ref[...] = (
        jnp.dot(b2_ref[...], wpb_ref[...], preferred_element_type=jnp.float32)
        + bp_ref[...]
    )


def _fold_weights(w2, wpb, b2, bp):
    h = w2.shape[0]
    return pl.pallas_call(
        _fold_body,
        out_shape=(
            jax.ShapeDtypeStruct((h, h), jnp.float32),
            jax.ShapeDtypeStruct((1, h), jnp.float32),
        ),
    )(w2, wpb, b2.reshape(1, h), bp.reshape(1, h))


_T = 256


def _fused_body(inp_ref, ps_ref, pe_ref, w1_ref, w2p_ref, wpa_ref, b1_ref,
                bpr_ref, out_ref):
    h = w2p_ref.shape[0]
    acc = jnp.dot(ps_ref[...], w1_ref[:h, :], preferred_element_type=jnp.float32)
    acc += jnp.dot(pe_ref[...], w1_ref[h:, :], preferred_element_type=jnp.float32)
    acc += b1_ref[...]
    acc = jnp.where(acc >= 0, acc, 0.01 * acc)
    out = jnp.dot(acc, w2p_ref[...], preferred_element_type=jnp.float32)
    out += jnp.dot(inp_ref[...], wpa_ref[...], preferred_element_type=jnp.float32)
    out_ref[...] = out + bpr_ref[...]


def _fused_chain(inp2, ps, pe, w1, w2p, wpa, b1, bpr):
    n, h = inp2.shape
    grid = (n // _T,)
    blk = lambda i: (i, 0)
    fixed = lambda i: (0, 0)
    return pl.pallas_call(
        _fused_body,
        grid=grid,
        in_specs=[
            pl.BlockSpec((_T, h), blk),
            pl.BlockSpec((_T, h), blk),
            pl.BlockSpec((_T, h), blk),
            pl.BlockSpec((2 * h, h), fixed),
            pl.BlockSpec((h, h), fixed),
            pl.BlockSpec((h, h), fixed),
            pl.BlockSpec((1, h), fixed),
            pl.BlockSpec((1, h), fixed),
        ],
        out_specs=pl.BlockSpec((_T, h), blk),
        out_shape=jax.ShapeDtypeStruct((n, h), jnp.float32),
        compiler_params=pltpu.CompilerParams(
            dimension_semantics=("arbitrary",),
        ),
    )(inp2, ps, pe, w1, w2p, wpa, b1, bpr)


_NCHUNKS = 2


def kernel(inp, pos_s, pos_e, pe_s, pe_e, W1, b1, W2, b2, Wp, bp):
    B, L, H = inp.shape
    n = B * L
    cn = n // _NCHUNKS
    inp2 = inp.reshape(n, H)
    idx_s = pos_s.reshape(n)
    idx_e = pos_e.reshape(n)
    w2p, bpr = _fold_weights(W2, Wp[H:], b2, bp)
    wpa = Wp[:H]
    b1r = b1.reshape(1, H)
    outs = []
    for c in range(_NCHUNKS):
        sl = slice(c * cn, (c + 1) * cn)
        ps, pe_g = _sc_gather_pair(pe_s, pe_e, idx_s[sl], idx_e[sl])
        outs.append(_fused_chain(inp2[sl], ps, pe_g, W1, w2p, wpa, b1r, bpr))
    out = jnp.concatenate(outs, axis=0)
    return out.reshape(B, L, H)


# speedup vs baseline: 1.2200x; 1.2200x over previous
import functools

import jax
import jax.numpy as jnp
from jax import lax
from jax.experimental import pallas as pl
from jax.experimental.pallas import tpu as pltpu
from jax.experimental.pallas import tpu_sc as plsc


_CHUNK = 64


def _sc_gather_pair(table_s, table_e, idx_s, idx_e):
    n = idx_s.shape[0]
    h = table_s.shape[1]
    info = plsc.get_sparse_core_info()
    nc, ns = info.num_cores, info.num_subcores
    nw = nc * ns
    per_w = n // nw
    nchunks = per_w // _CHUNK
    mesh = plsc.VectorSubcoreMesh(core_axis_name="core", subcore_axis_name="subcore")

    @functools.partial(
        pl.kernel,
        out_type=(
            jax.ShapeDtypeStruct((n, h), jnp.float32),
            jax.ShapeDtypeStruct((n, h), jnp.float32),
        ),
        mesh=mesh,
        scratch_types=[
            pltpu.VMEM((per_w,), jnp.int32),
            pltpu.VMEM((_CHUNK, h), jnp.float32),
            pltpu.SemaphoreType.DMA,
        ],
    )
    def gather_kernel(ts_hbm, te_hbm, is_hbm, ie_hbm, os_hbm, oe_hbm,
                      idx_v, rows_v, sem):
        wid = lax.axis_index("subcore") * nc + lax.axis_index("core")
        base = wid * per_w

        def one_table(t_hbm, i_hbm, o_hbm):
            pltpu.sync_copy(i_hbm.at[pl.ds(base, per_w)], idx_v)

            @pl.loop(0, nchunks)
            def _(c):
                off = c * _CHUNK
                pltpu.async_copy(
                    t_hbm.at[idx_v.at[pl.ds(off, _CHUNK)]], rows_v, sem
                ).wait()
                pltpu.sync_copy(rows_v, o_hbm.at[pl.ds(base + off, _CHUNK)])

        one_table(ts_hbm, is_hbm, os_hbm)
        one_table(te_hbm, ie_hbm, oe_hbm)

    return gather_kernel(table_s, table_e, idx_s, idx_e)


def _fold_body(w2_ref, wpb_ref, b2_ref, bp_ref, w2p_ref, bpr_ref):
    w2p_ref[...] = jnp.dot(
        w2_ref[...], wpb_ref[...], preferred_element_type=jnp.float32
    )
    bpr_ref[...] = (
        jnp.dot(b2_ref[...], wpb_ref[...], preferred_element_type=jnp.float32)
        + bp_ref[...]
    )


def _fold_weights(w2, wpb, b2, bp):
    h = w2.shape[0]
    return pl.pallas_call(
        _fold_body,
        out_shape=(
            jax.ShapeDtypeStruct((h, h), jnp.float32),
            jax.ShapeDtypeStruct((1, h), jnp.float32),
        ),
    )(w2, wpb, b2.reshape(1, h), bp.reshape(1, h))


_T = 256


def _acc0_body(inp_ref, wpa_ref, bpr_ref, out_ref):
    out_ref[...] = (
        jnp.dot(inp_ref[...], wpa_ref[...], preferred_element_type=jnp.float32)
        + bpr_ref[...]
    )


def _acc0(inp2, wpa, bpr):
    n, h = inp2.shape
    return pl.pallas_call(
        _acc0_body,
        grid=(n // _T,),
        in_specs=[
            pl.BlockSpec((_T, h), lambda i: (i, 0)),
            pl.BlockSpec((h, h), lambda i: (0, 0)),
            pl.BlockSpec((1, h), lambda i: (0, 0)),
        ],
        out_specs=pl.BlockSpec((_T, h), lambda i: (i, 0)),
        out_shape=jax.ShapeDtypeStruct((n, h), jnp.float32),
        compiler_params=pltpu.CompilerParams(
            dimension_semantics=("arbitrary",),
        ),
    )(inp2, wpa, bpr)


def _fused_body(ps_ref, pe_ref, acc0_ref, w1_ref, w2p_ref, b1_ref, out_ref):
    h = w2p_ref.shape[0]
    acc = jnp.dot(ps_ref[...], w1_ref[:h, :], preferred_element_type=jnp.float32)
    acc += jnp.dot(pe_ref[...], w1_ref[h:, :], preferred_element_type=jnp.float32)
    acc += b1_ref[...]
    acc = jnp.where(acc >= 0, acc, 0.01 * acc)
    out_ref[...] = acc0_ref[...] + jnp.dot(
        acc, w2p_ref[...], preferred_element_type=jnp.float32
    )


def _fused_chain(ps, pe, acc0, w1, w2p, b1):
    n, h = ps.shape
    blk = lambda i: (i, 0)
    fixed = lambda i: (0, 0)
    return pl.pallas_call(
        _fused_body,
        grid=(n // _T,),
        in_specs=[
            pl.BlockSpec((_T, h), blk),
            pl.BlockSpec((_T, h), blk),
            pl.BlockSpec((_T, h), blk),
            pl.BlockSpec((2 * h, h), fixed),
            pl.BlockSpec((h, h), fixed),
            pl.BlockSpec((1, h), fixed),
        ],
        out_specs=pl.BlockSpec((_T, h), blk),
        out_shape=jax.ShapeDtypeStruct((n, h), jnp.float32),
        compiler_params=pltpu.CompilerParams(
            dimension_semantics=("arbitrary",),
        ),
    )(ps, pe, acc0, w1, w2p, b1)


def kernel(inp, pos_s, pos_e, pe_s, pe_e, W1, b1, W2, b2, Wp, bp):
    B, L, H = inp.shape
    n = B * L
    inp2 = inp.reshape(n, H)
    ps, pe_g = _sc_gather_pair(pe_s, pe_e, pos_s.reshape(n), pos_e.reshape(n))
    w2p, bpr = _fold_weights(W2, Wp[H:], b2, bp)
    acc0 = _acc0(inp2, Wp[:H], bpr)
    out = _fused_chain(ps, pe_g, acc0, W1, w2p, b1.reshape(1, H))
    return out.reshape(B, L, H)


# speedup vs baseline: 1.3786x; 1.1300x over previous
import functools

import jax
import jax.numpy as jnp
from jax import lax
from jax.experimental import pallas as pl
from jax.experimental.pallas import tpu as pltpu
from jax.experimental.pallas import tpu_sc as plsc


_CHUNK = 64


def _sc_gather_pair(table_s, table_e, idx_s, idx_e):
    n = idx_s.shape[0]
    h = table_s.shape[1]
    info = plsc.get_sparse_core_info()
    nc, ns = info.num_cores, info.num_subcores
    nw = nc * ns
    per_w = n // nw
    nchunks = per_w // _CHUNK
    mesh = plsc.VectorSubcoreMesh(core_axis_name="core", subcore_axis_name="subcore")

    @functools.partial(
        pl.kernel,
        out_type=(
            jax.ShapeDtypeStruct((n, h), jnp.float32),
            jax.ShapeDtypeStruct((n, h), jnp.float32),
        ),
        mesh=mesh,
        scratch_types=[
            pltpu.VMEM((per_w,), jnp.int32),
            pltpu.VMEM((_CHUNK, h), jnp.float32),
            pltpu.SemaphoreType.DMA,
        ],
    )
    def gather_kernel(ts_hbm, te_hbm, is_hbm, ie_hbm, os_hbm, oe_hbm,
                      idx_v, rows_v, sem):
        wid = lax.axis_index("subcore") * nc + lax.axis_index("core")
        base = wid * per_w

        def one_table(t_hbm, i_hbm, o_hbm):
            pltpu.sync_copy(i_hbm.at[pl.ds(base, per_w)], idx_v)

            @pl.loop(0, nchunks)
            def _(c):
                off = c * _CHUNK
                pltpu.async_copy(
                    t_hbm.at[idx_v.at[pl.ds(off, _CHUNK)]], rows_v, sem
                ).wait()
                pltpu.sync_copy(rows_v, o_hbm.at[pl.ds(base + off, _CHUNK)])

        one_table(ts_hbm, is_hbm, os_hbm)
        one_table(te_hbm, ie_hbm, oe_hbm)

    return gather_kernel(table_s, table_e, idx_s, idx_e)


def _fold_body(w2_ref, wpb_ref, b2_ref, bp_ref, w2p_ref, bpr_ref):
    w2p_ref[...] = jnp.dot(
        w2_ref[...], wpb_ref[...], preferred_element_type=jnp.float32
    )
    bpr_ref[...] = (
        jnp.dot(b2_ref[...], wpb_ref[...], preferred_element_type=jnp.float32)
        + bp_ref[...]
    )


def _fold_weights(w2, wpb, b2, bp):
    h = w2.shape[0]
    return pl.pallas_call(
        _fold_body,
        out_shape=(
            jax.ShapeDtypeStruct((h, h), jnp.float32),
            jax.ShapeDtypeStruct((1, h), jnp.float32),
        ),
    )(w2, wpb, b2.reshape(1, h), bp.reshape(1, h))


_T = 512


def _fused_body(inp_ref, ps_ref, pe_ref, w1_ref, w2p_ref, wpa_ref, b1_ref,
                bpr_ref, out_ref):
    h = w2p_ref.shape[0]
    acc = jnp.dot(ps_ref[...], w1_ref[:h, :], preferred_element_type=jnp.float32)
    acc += jnp.dot(pe_ref[...], w1_ref[h:, :], preferred_element_type=jnp.float32)
    acc += b1_ref[...]
    acc = jnp.where(acc >= 0, acc, 0.01 * acc)
    out = jnp.dot(acc, w2p_ref[...], preferred_element_type=jnp.float32)
    out += jnp.dot(inp_ref[...], wpa_ref[...], preferred_element_type=jnp.float32)
    out_ref[...] = out + bpr_ref[...]


def _fused_chain(inp2, ps, pe, w1, w2p, wpa, b1, bpr):
    n, h = inp2.shape
    blk = lambda i: (i, 0)
    fixed = lambda i: (0, 0)
    return pl.pallas_call(
        _fused_body,
        grid=(n // _T,),
        in_specs=[
            pl.BlockSpec((_T, h), blk),
            pl.BlockSpec((_T, h), blk),
            pl.BlockSpec((_T, h), blk),
            pl.BlockSpec((2 * h, h), fixed),
            pl.BlockSpec((h, h), fixed),
            pl.BlockSpec((h, h), fixed),
            pl.BlockSpec((1, h), fixed),
            pl.BlockSpec((1, h), fixed),
        ],
        out_specs=pl.BlockSpec((_T, h), blk),
        out_shape=jax.ShapeDtypeStruct((n, h), jnp.float32),
        compiler_params=pltpu.CompilerParams(
            dimension_semantics=("arbitrary",),
        ),
    )(inp2, ps, pe, w1, w2p, wpa, b1, bpr)


def kernel(inp, pos_s, pos_e, pe_s, pe_e, W1, b1, W2, b2, Wp, bp):
    B, L, H = inp.shape
    n = B * L
    inp2 = inp.reshape(n, H)
    ps, pe_g = _sc_gather_pair(pe_s, pe_e, pos_s.reshape(n), pos_e.reshape(n))
    w2p, bpr = _fold_weights(W2, Wp[H:], b2, bp)
    out = _fused_chain(inp2, ps, pe_g, W1, w2p, Wp[:H], b1.reshape(1, H), bpr)
    return out.reshape(B, L, H)


# speedup vs baseline: 1.3818x; 1.0023x over previous
import functools

import jax
import jax.numpy as jnp
from jax import lax
from jax.experimental import pallas as pl
from jax.experimental.pallas import tpu as pltpu
from jax.experimental.pallas import tpu_sc as plsc


_CHUNK = 32


def _sc_gather_pair(table_s, table_e, idx_s, idx_e):
    n = idx_s.shape[0]
    h = table_s.shape[1]
    info = plsc.get_sparse_core_info()
    nc, ns = info.num_cores, info.num_subcores
    nw = nc * ns
    per_w = n // nw
    nchunks = per_w // _CHUNK
    mesh = plsc.VectorSubcoreMesh(core_axis_name="core", subcore_axis_name="subcore")

    @functools.partial(
        pl.kernel,
        out_type=(
            jax.ShapeDtypeStruct((n, h), jnp.float32),
            jax.ShapeDtypeStruct((n, h), jnp.float32),
        ),
        mesh=mesh,
        scratch_types=[
            pltpu.VMEM((per_w,), jnp.int32),
            pltpu.VMEM((per_w,), jnp.int32),
            pltpu.VMEM((_CHUNK, h), jnp.float32),
            pltpu.VMEM((_CHUNK, h), jnp.float32),
            pltpu.SemaphoreType.DMA,
            pltpu.SemaphoreType.DMA,
            pltpu.SemaphoreType.DMA,
        ],
    )
    def gather_kernel(ts_hbm, te_hbm, is_hbm, ie_hbm, os_hbm, oe_hbm,
                      idx_s_v, idx_e_v, rows0, rows1, gsem, wsem0, wsem1):
        wid = lax.axis_index("subcore") * nc + lax.axis_index("core")
        base = wid * per_w
        pltpu.sync_copy(is_hbm.at[pl.ds(base, per_w)], idx_s_v)
        pltpu.sync_copy(ie_hbm.at[pl.ds(base, per_w)], idx_e_v)

        rows = (rows0, rows1)
        wsems = (wsem0, wsem1)
        pending = [None, None]
        k = 0
        for t_hbm, i_v, o_hbm in ((ts_hbm, idx_s_v, os_hbm),
                                  (te_hbm, idx_e_v, oe_hbm)):
            for c in range(nchunks):
                b = k % 2
                if pending[b] is not None:
                    pending[b].wait()
                off = c * _CHUNK
                pltpu.async_copy(
                    t_hbm.at[i_v.at[pl.ds(off, _CHUNK)]], rows[b], gsem
                ).wait()
                pending[b] = pltpu.async_copy(
                    rows[b], o_hbm.at[pl.ds(base + off, _CHUNK)], wsems[b]
                )
                k += 1
        for p in pending:
            if p is not None:
                p.wait()

    return gather_kernel(table_s, table_e, idx_s, idx_e)


def _fold_body(w2_ref, wpb_ref, b2_ref, bp_ref, w2p_ref, bpr_ref):
    w2p_ref[...] = jnp.dot(
        w2_ref[...], wpb_ref[...], preferred_element_type=jnp.float32
    )
    bpr_ref[...] = (
        jnp.dot(b2_ref[...], wpb_ref[...], preferred_element_type=jnp.float32)
        + bp_ref[...]
    )


def _fold_weights(w2, wpb, b2, bp):
    h = w2.shape[0]
    return pl.pallas_call(
        _fold_body,
        out_shape=(
            jax.ShapeDtypeStruct((h, h), jnp.float32),
            jax.ShapeDtypeStruct((1, h), jnp.float32),
        ),
    )(w2, wpb, b2.reshape(1, h), bp.reshape(1, h))


_T = 512


def _fused_body(inp_ref, ps_ref, pe_ref, w1_ref, w2p_ref, wpa_ref, b1_ref,
                bpr_ref, out_ref):
    h = w2p_ref.shape[0]
    acc = jnp.dot(ps_ref[...], w1_ref[:h, :], preferred_element_type=jnp.float32)
    acc += jnp.dot(pe_ref[...], w1_ref[h:, :], preferred_element_type=jnp.float32)
    acc += b1_ref[...]
    acc = jnp.where(acc >= 0, acc, 0.01 * acc)
    out = jnp.dot(acc, w2p_ref[...], preferred_element_type=jnp.float32)
    out += jnp.dot(inp_ref[...], wpa_ref[...], preferred_element_type=jnp.float32)
    out_ref[...] = out + bpr_ref[...]


def _fused_chain(inp2, ps, pe, w1, w2p, wpa, b1, bpr):
    n, h = inp2.shape
    blk = lambda i: (i, 0)
    fixed = lambda i: (0, 0)
    return pl.pallas_call(
        _fused_body,
        grid=(n // _T,),
        in_specs=[
            pl.BlockSpec((_T, h), blk),
            pl.BlockSpec((_T, h), blk),
            pl.BlockSpec((_T, h), blk),
            pl.BlockSpec((2 * h, h), fixed),
            pl.BlockSpec((h, h), fixed),
            pl.BlockSpec((h, h), fixed),
            pl.BlockSpec((1, h), fixed),
            pl.BlockSpec((1, h), fixed),
        ],
        out_specs=pl.BlockSpec((_T, h), blk),
        out_shape=jax.ShapeDtypeStruct((n, h), jnp.float32),
        compiler_params=pltpu.CompilerParams(
            dimension_semantics=("arbitrary",),
        ),
    )(inp2, ps, pe, w1, w2p, wpa, b1, bpr)


def kernel(inp, pos_s, pos_e, pe_s, pe_e, W1, b1, W2, b2, Wp, bp):
    B, L, H = inp.shape
    n = B * L
    inp2 = inp.reshape(n, H)
    ps, pe_g = _sc_gather_pair(pe_s, pe_e, pos_s.reshape(n), pos_e.reshape(n))
    w2p, bpr = _fold_weights(W2, Wp[H:], b2, bp)
    out = _fused_chain(inp2, ps, pe_g, W1, w2p, Wp[:H], b1.reshape(1, H), bpr)
    return out.reshape(B, L, H)


# speedup vs baseline: 1.3907x; 1.0064x over previous
import functools

import jax
import jax.numpy as jnp
from jax import lax
from jax.experimental import pallas as pl
from jax.experimental.pallas import tpu as pltpu
from jax.experimental.pallas import tpu_sc as plsc


_CHUNK = 32


def _sc_gather_pair(table_s, table_e, idx_s, idx_e):
    n = idx_s.shape[0]
    h = table_s.shape[1]
    info = plsc.get_sparse_core_info()
    nc, ns = info.num_cores, info.num_subcores
    nw = nc * ns
    per_w = n // nw
    nchunks = per_w // _CHUNK
    mesh = plsc.VectorSubcoreMesh(core_axis_name="core", subcore_axis_name="subcore")

    @functools.partial(
        pl.kernel,
        out_type=(
            jax.ShapeDtypeStruct((n, h), jnp.float32),
            jax.ShapeDtypeStruct((n, h), jnp.float32),
        ),
        mesh=mesh,
        scratch_types=[
            pltpu.VMEM((per_w,), jnp.int32),
            pltpu.VMEM((per_w,), jnp.int32),
            pltpu.VMEM((_CHUNK, h), jnp.float32),
            pltpu.VMEM((_CHUNK, h), jnp.float32),
            pltpu.SemaphoreType.DMA,
            pltpu.SemaphoreType.DMA,
            pltpu.SemaphoreType.DMA,
        ],
    )
    def gather_kernel(ts_hbm, te_hbm, is_hbm, ie_hbm, os_hbm, oe_hbm,
                      idx_s_v, idx_e_v, rows0, rows1, gsem, wsem0, wsem1):
        wid = lax.axis_index("subcore") * nc + lax.axis_index("core")
        base = wid * per_w
        pltpu.sync_copy(is_hbm.at[pl.ds(base, per_w)], idx_s_v)
        pltpu.sync_copy(ie_hbm.at[pl.ds(base, per_w)], idx_e_v)

        rows = (rows0, rows1)
        wsems = (wsem0, wsem1)
        pending = [None, None]
        k = 0
        for t_hbm, i_v, o_hbm in ((ts_hbm, idx_s_v, os_hbm),
                                  (te_hbm, idx_e_v, oe_hbm)):
            for c in range(nchunks):
                b = k % 2
                if pending[b] is not None:
                    pending[b].wait()
                off = c * _CHUNK
                pltpu.async_copy(
                    t_hbm.at[i_v.at[pl.ds(off, _CHUNK)]], rows[b], gsem
                ).wait()
                pending[b] = pltpu.async_copy(
                    rows[b], o_hbm.at[pl.ds(base + off, _CHUNK)], wsems[b]
                )
                k += 1
        for p in pending:
            if p is not None:
                p.wait()

    return gather_kernel(table_s, table_e, idx_s, idx_e)


_T = 512


def _fused_body(inp_ref, ps_ref, pe_ref, w1_ref, w2_ref, wp_ref, b1_ref,
                b2_ref, bp_ref, out_ref, w2p_s, bpr_s):
    h = w2_ref.shape[0]

    @pl.when(pl.program_id(0) == 0)
    def _():
        w2p_s[...] = jnp.dot(
            w2_ref[...], wp_ref[h:, :], preferred_element_type=jnp.float32
        )
        bpr_s[...] = (
            jnp.dot(b2_ref[...], wp_ref[h:, :], preferred_element_type=jnp.float32)
            + bp_ref[...]
        )

    acc = jnp.dot(ps_ref[...], w1_ref[:h, :], preferred_element_type=jnp.float32)
    acc += jnp.dot(pe_ref[...], w1_ref[h:, :], preferred_element_type=jnp.float32)
    acc += b1_ref[...]
    acc = jnp.where(acc >= 0, acc, 0.01 * acc)
    out = jnp.dot(acc, w2p_s[...], preferred_element_type=jnp.float32)
    out += jnp.dot(inp_ref[...], wp_ref[:h, :], preferred_element_type=jnp.float32)
    out_ref[...] = out + bpr_s[...]


def _fused_chain(inp2, ps, pe, w1, w2, wp, b1, b2, bp):
    n, h = inp2.shape
    blk = lambda i: (i, 0)
    fixed = lambda i: (0, 0)
    return pl.pallas_call(
        _fused_body,
        grid=(n // _T,),
        in_specs=[
            pl.BlockSpec((_T, h), blk),
            pl.BlockSpec((_T, h), blk),
            pl.BlockSpec((_T, h), blk),
            pl.BlockSpec((2 * h, h), fixed),
            pl.BlockSpec((h, h), fixed),
            pl.BlockSpec((2 * h, h), fixed),
            pl.BlockSpec((1, h), fixed),
            pl.BlockSpec((1, h), fixed),
            pl.BlockSpec((1, h), fixed),
        ],
        out_specs=pl.BlockSpec((_T, h), blk),
        out_shape=jax.ShapeDtypeStruct((n, h), jnp.float32),
        scratch_shapes=[
            pltpu.VMEM((h, h), jnp.float32),
            pltpu.VMEM((1, h), jnp.float32),
        ],
        compiler_params=pltpu.CompilerParams(
            dimension_semantics=("arbitrary",),
        ),
    )(inp2, ps, pe, w1, w2, wp, b1, b2, bp)


def kernel(inp, pos_s, pos_e, pe_s, pe_e, W1, b1, W2, b2, Wp, bp):
    B, L, H = inp.shape
    n = B * L
    inp2 = inp.reshape(n, H)
    ps, pe_g = _sc_gather_pair(pe_s, pe_e, pos_s.reshape(n), pos_e.reshape(n))
    out = _fused_chain(
        inp2, ps, pe_g, W1, W2, Wp,
        b1.reshape(1, H), b2.reshape(1, H), bp.reshape(1, H)
    )
    return out.reshape(B, L, H)


# speedup vs baseline: 1.4267x; 1.0259x over previous
import functools

import jax
import jax.numpy as jnp
from jax import lax
from jax.experimental import pallas as pl
from jax.experimental.pallas import tpu as pltpu
from jax.experimental.pallas import tpu_sc as plsc


_CHUNK = 32


def _sc_gather_pair(table_s, table_e, idx_s, idx_e):
    n = idx_s.shape[0]
    h = table_s.shape[1]
    info = plsc.get_sparse_core_info()
    nc, ns = info.num_cores, info.num_subcores
    nw = nc * ns
    per_w = n // nw
    nchunks = per_w // _CHUNK
    mesh = plsc.VectorSubcoreMesh(core_axis_name="core", subcore_axis_name="subcore")

    @functools.partial(
        pl.kernel,
        out_type=(
            jax.ShapeDtypeStruct((n, h), jnp.float32),
            jax.ShapeDtypeStruct((n, h), jnp.float32),
        ),
        mesh=mesh,
        scratch_types=[
            pltpu.VMEM((per_w,), jnp.int32),
            pltpu.VMEM((per_w,), jnp.int32),
            pltpu.VMEM((_CHUNK, h), jnp.float32),
            pltpu.VMEM((_CHUNK, h), jnp.float32),
            pltpu.SemaphoreType.DMA,
            pltpu.SemaphoreType.DMA,
            pltpu.SemaphoreType.DMA,
            pltpu.SemaphoreType.DMA,
        ],
    )
    def gather_kernel(ts_hbm, te_hbm, is_hbm, ie_hbm, os_hbm, oe_hbm,
                      idx_s_v, idx_e_v, rows0, rows1, gsem0, gsem1,
                      wsem0, wsem1):
        wid = lax.axis_index("subcore") * nc + lax.axis_index("core")
        base = wid * per_w
        i0 = pltpu.async_copy(is_hbm.at[pl.ds(base, per_w)], idx_s_v, gsem0)
        i1 = pltpu.async_copy(ie_hbm.at[pl.ds(base, per_w)], idx_e_v, gsem1)
        i0.wait()
        i1.wait()

        rows = (rows0, rows1)
        gsems = (gsem0, gsem1)
        wsems = (wsem0, wsem1)
        total = 2 * nchunks

        def chunk_src(k):
            if k < nchunks:
                return ts_hbm, idx_s_v, os_hbm, k * _CHUNK
            c = k - nchunks
            return te_hbm, idx_e_v, oe_hbm, c * _CHUNK

        gpend = [None, None]
        wpend = [None, None]
        for k in range(total + 1):
            if k < total:
                b = k % 2
                if wpend[b] is not None:
                    wpend[b].wait()
                t_hbm, i_v, _, off = chunk_src(k)
                gpend[b] = pltpu.async_copy(
                    t_hbm.at[i_v.at[pl.ds(off, _CHUNK)]], rows[b], gsems[b]
                )
            if k >= 1:
                b = (k - 1) % 2
                gpend[b].wait()
                _, _, o_hbm, off = chunk_src(k - 1)
                wpend[b] = pltpu.async_copy(
                    rows[b], o_hbm.at[pl.ds(base + off, _CHUNK)], wsems[b]
                )
        for p in wpend:
            if p is not None:
                p.wait()

    return gather_kernel(table_s, table_e, idx_s, idx_e)


_T = 512


def _fused_body(inp_ref, ps_ref, pe_ref, w1_ref, w2_ref, wp_ref, b1_ref,
                b2_ref, bp_ref, out_ref, w2p_s, bpr_s):
    h = w2_ref.shape[0]

    @pl.when(pl.program_id(0) == 0)
    def _():
        w2p_s[...] = jnp.dot(
            w2_ref[...], wp_ref[h:, :], preferred_element_type=jnp.float32
        )
        bpr_s[...] = (
            jnp.dot(b2_ref[...], wp_ref[h:, :], preferred_element_type=jnp.float32)
            + bp_ref[...]
        )

    acc = jnp.dot(ps_ref[...], w1_ref[:h, :], preferred_element_type=jnp.float32)
    acc += jnp.dot(pe_ref[...], w1_ref[h:, :], preferred_element_type=jnp.float32)
    acc += b1_ref[...]
    acc = jnp.where(acc >= 0, acc, 0.01 * acc)
    out = jnp.dot(acc, w2p_s[...], preferred_element_type=jnp.float32)
    out += jnp.dot(inp_ref[...], wp_ref[:h, :], preferred_element_type=jnp.float32)
    out_ref[...] = out + bpr_s[...]


def _fused_chain(inp2, ps, pe, w1, w2, wp, b1, b2, bp):
    n, h = inp2.shape
    blk = lambda i: (i, 0)
    fixed = lambda i: (0, 0)
    return pl.pallas_call(
        _fused_body,
        grid=(n // _T,),
        in_specs=[
            pl.BlockSpec((_T, h), blk),
            pl.BlockSpec((_T, h), blk),
            pl.BlockSpec((_T, h), blk),
            pl.BlockSpec((2 * h, h), fixed),
            pl.BlockSpec((h, h), fixed),
            pl.BlockSpec((2 * h, h), fixed),
            pl.BlockSpec((1, h), fixed),
            pl.BlockSpec((1, h), fixed),
            pl.BlockSpec((1, h), fixed),
        ],
        out_specs=pl.BlockSpec((_T, h), blk),
        out_shape=jax.ShapeDtypeStruct((n, h), jnp.float32),
        scratch_shapes=[
            pltpu.VMEM((h, h), jnp.float32),
            pltpu.VMEM((1, h), jnp.float32),
        ],
        compiler_params=pltpu.CompilerParams(
            dimension_semantics=("arbitrary",),
        ),
    )(inp2, ps, pe, w1, w2, wp, b1, b2, bp)


def kernel(inp, pos_s, pos_e, pe_s, pe_e, W1, b1, W2, b2, Wp, bp):
    B, L, H = inp.shape
    n = B * L
    inp2 = inp.reshape(n, H)
    ps, pe_g = _sc_gather_pair(pe_s, pe_e, pos_s.reshape(n), pos_e.reshape(n))
    out = _fused_chain(
        inp2, ps, pe_g, W1, W2, Wp,
        b1.reshape(1, H), b2.reshape(1, H), bp.reshape(1, H)
    )
    return out.reshape(B, L, H)
